# Initial kernel scaffold; baseline (speedup 1.0000x reference)
import functools

import jax
import jax.numpy as jnp
from jax import lax
from jax.experimental import pallas as pl
from jax.experimental.pallas import tpu as pltpu
from jax.experimental.pallas import tpu_sc as plsc


_L = 16


def _sc_geometry():
    info = plsc.get_sparse_core_info()
    return info.num_cores, info.num_subcores


def _make_layer(n_nodes, d, n_edges, nc, ns):
    nw = nc * ns
    e_per_w = n_edges // nw
    chunk = 80
    n_chunks = e_per_w // chunk
    rows_per_tile = n_nodes // ns
    zrows = 125
    n_zblocks = rows_per_tile // zrows
    mesh = plsc.VectorSubcoreMesh(core_axis_name="c", subcore_axis_name="s")

    @functools.partial(
        pl.kernel,
        mesh=mesh,
        out_type=jax.ShapeDtypeStruct((nc * n_nodes, d), jnp.float32),
        scratch_types=[
            pltpu.VMEM((e_per_w,), jnp.int32),
            pltpu.VMEM((n_chunks, chunk), jnp.int32),
            pltpu.VMEM((e_per_w,), jnp.float32),
            pltpu.VMEM((chunk, d), jnp.float32),
            pltpu.VMEM((zrows, d), jnp.float32),
            pltpu.VMEM_SHARED((n_nodes, d), jnp.float32),
            pltpu.SemaphoreType.DMA,
        ],
    )
    def layer(in_hbm, src_hbm, dst_hbm, w_hbm, out_hbm,
              src_v, dst_v, w_v, rows_v, zero_v, acc_sh, sem):
        cid = lax.axis_index("c")
        sid = lax.axis_index("s")
        wid = sid * nc + cid

        pltpu.sync_copy(src_hbm.at[wid], src_v)
        pltpu.sync_copy(dst_hbm.at[wid], dst_v)
        pltpu.sync_copy(w_hbm.at[wid], w_v)

        zvec = jnp.zeros((_L,), jnp.float32)

        def zfill(i, _):
            for j in range(d // _L):
                zero_v[i, pl.ds(j * _L, _L)] = zvec
            return 0

        lax.fori_loop(0, zrows, zfill, 0)
        r0 = sid * rows_per_tile
        for z in range(n_zblocks):
            pltpu.sync_copy(zero_v, acc_sh.at[pl.ds(r0 + z * zrows, zrows)])
        plsc.subcore_barrier()

        def chunk_body(k, _):
            idx = src_v.at[pl.ds(k * chunk, chunk)]
            pltpu.async_copy(in_hbm.at[idx], rows_v, sem).wait()

            def scale(e, _):
                we = w_v[k * chunk + e]
                for j in range(d // _L):
                    sl = pl.ds(j * _L, _L)
                    rows_v[e, sl] = rows_v[e, sl] * we
                return 0

            lax.fori_loop(0, chunk, scale, 0)
            pltpu.sync_copy(rows_v, acc_sh.at[dst_v.at[k]], add=True)
            return 0

        lax.fori_loop(0, n_chunks, chunk_body, 0)
        plsc.subcore_barrier()

        pltpu.sync_copy(
            acc_sh.at[pl.ds(r0, rows_per_tile)],
            out_hbm.at[pl.ds(cid * n_nodes + r0, rows_per_tile)],
        )

    return layer


def _combine(partials, acc, n_nodes, d):
    br = 1000
    grid = n_nodes // br

    def body(p_ref, acc_ref, e_ref, accout_ref):
        e = p_ref[0] + p_ref[1]
        e_ref[...] = e
        accout_ref[...] = acc_ref[...] + e

    p3 = partials.reshape(2, n_nodes, d)
    return pl.pallas_call(
        body,
        grid=(grid,),
        in_specs=[
            pl.BlockSpec((2, br, d), lambda i: (0, i, 0)),
            pl.BlockSpec((br, d), lambda i: (i, 0)),
        ],
        out_specs=[
            pl.BlockSpec((br, d), lambda i: (i, 0)),
            pl.BlockSpec((br, d), lambda i: (i, 0)),
        ],
        out_shape=[jax.ShapeDtypeStruct((n_nodes, d), jnp.float32)] * 2,
    )(p3, acc)


def _combine_last(partials, acc, n_nodes, d):
    br = 1000
    grid = n_nodes // br

    def body(p_ref, acc_ref, accout_ref):
        accout_ref[...] = acc_ref[...] + p_ref[0] + p_ref[1]

    p3 = partials.reshape(2, n_nodes, d)
    return pl.pallas_call(
        body,
        grid=(grid,),
        in_specs=[
            pl.BlockSpec((2, br, d), lambda i: (0, i, 0)),
            pl.BlockSpec((br, d), lambda i: (i, 0)),
        ],
        out_specs=pl.BlockSpec((br, d), lambda i: (i, 0)),
        out_shape=jax.ShapeDtypeStruct((n_nodes, d), jnp.float32),
    )(p3, acc)


def _make_final(n_nodes, d, b, nc, ns):
    nw = nc * ns
    per_w = b // nw
    mesh = plsc.VectorSubcoreMesh(core_axis_name="c", subcore_axis_name="s")

    @functools.partial(
        pl.kernel,
        mesh=mesh,
        out_type=jax.ShapeDtypeStruct((b,), jnp.float32),
        scratch_types=[
            pltpu.VMEM((per_w,), jnp.int32),
            pltpu.VMEM((per_w,), jnp.int32),
            pltpu.VMEM((per_w, d), jnp.float32),
            pltpu.VMEM((per_w, d), jnp.float32),
            pltpu.VMEM((per_w,), jnp.float32),
            pltpu.SemaphoreType.DMA,
        ],
    )
    def final(acc_hbm, users_hbm, items_hbm, out_hbm,
              ui_v, ii_v, ur_v, ir_v, g_v, sem):
        cid = lax.axis_index("c")
        sid = lax.axis_index("s")
        wid = sid * nc + cid
        b0 = wid * per_w
        pltpu.sync_copy(users_hbm.at[pl.ds(b0, per_w)], ui_v)
        pltpu.sync_copy(items_hbm.at[pl.ds(b0, per_w)], ii_v)
        pltpu.async_copy(acc_hbm.at[ui_v], ur_v, sem).wait()
        pltpu.async_copy(acc_hbm.at[ii_v], ir_v, sem).wait()

        def dot(p, _):
            s = jnp.zeros((_L,), jnp.float32)
            for j in range(d // _L):
                sl = pl.ds(j * _L, _L)
                s = s + ur_v[p, sl] * ir_v[p, sl]
            g_v[p] = jnp.sum(s) * jnp.float32(1.0 / 16.0)
            return 0

        lax.fori_loop(0, per_w, dot, 0)
        pltpu.sync_copy(g_v, out_hbm.at[pl.ds(b0, per_w)])

    return final


def kernel(emb, edge_weight, edge_index, users, items):
    n_nodes, d = emb.shape
    n_edges = edge_weight.shape[0]
    b = users.shape[0]
    nc, ns = _sc_geometry()
    nw = nc * ns
    e_per_w = n_edges // nw
    chunk = 80
    n_chunks = e_per_w // chunk

    src = edge_index[0].reshape(nw, e_per_w)
    dst = edge_index[1].reshape(nw, n_chunks, chunk)
    w = edge_weight.reshape(nw, e_per_w)

    layer = _make_layer(n_nodes, d, n_edges, nc, ns)
    final = _make_final(n_nodes, d, b, nc, ns)

    p1 = layer(emb, src, dst, w)
    e1, acc = _combine(p1, emb, n_nodes, d)
    p2 = layer(e1, src, dst, w)
    e2, acc = _combine(p2, acc, n_nodes, d)
    p3 = layer(e2, src, dst, w)
    acc = _combine_last(p3, acc, n_nodes, d)
    return final(acc, users, items)


# speedup vs baseline: 5.6014x; 5.6014x over previous
import functools

import jax
import jax.numpy as jnp
from jax import lax
from jax.experimental import pallas as pl
from jax.experimental.pallas import tpu as pltpu
from jax.experimental.pallas import tpu_sc as plsc


_L = 16


def _sc_geometry():
    info = plsc.get_sparse_core_info()
    return info.num_cores, info.num_subcores


def _make_layer(n_nodes, d, n_edges, nc, ns):
    nw = nc * ns
    e_per_w = n_edges // nw
    chunk = 80
    n_chunks = e_per_w // chunk
    nzt = 10
    rows_per_zt = n_nodes // nzt
    mesh = plsc.VectorSubcoreMesh(core_axis_name="c", subcore_axis_name="s")

    @functools.partial(
        pl.kernel,
        mesh=mesh,
        out_type=jax.ShapeDtypeStruct((nc * n_nodes, d), jnp.float32),
        scratch_types=[
            pltpu.VMEM((e_per_w,), jnp.int32),
            pltpu.VMEM((n_chunks, chunk), jnp.int32),
            pltpu.VMEM((e_per_w,), jnp.float32),
            pltpu.VMEM((chunk, d), jnp.float32),
            pltpu.VMEM_SHARED((n_nodes, d), jnp.float32),
            pltpu.SemaphoreType.DMA,
        ],
    )
    def layer(in_hbm, src_hbm, dst_hbm, w_hbm, out_hbm,
              src_v, dst_v, w_v, rows_v, acc_sh, sem):
        cid = lax.axis_index("c")
        sid = lax.axis_index("s")
        wid = sid * nc + cid
        ebase = pl.multiple_of(wid * e_per_w, 8)

        pltpu.sync_copy(src_hbm.at[pl.ds(ebase, e_per_w)], src_v)
        pltpu.sync_copy(dst_hbm.at[wid], dst_v)
        pltpu.sync_copy(w_hbm.at[pl.ds(ebase, e_per_w)], w_v)

        zvec = jnp.zeros((_L,), jnp.float32)

        def zfill(i, _):
            for j in range(d // _L):
                rows_v[i, pl.ds(j * _L, _L)] = zvec
            return 0

        lax.fori_loop(0, chunk, zfill, 0)
        r0 = pl.multiple_of(sid * rows_per_zt, 8)

        @pl.when(sid < nzt)
        def _zero():
            for z in range(rows_per_zt // chunk):
                pltpu.sync_copy(rows_v,
                                acc_sh.at[pl.ds(r0 + z * chunk, chunk)])
            rem = rows_per_zt % chunk
            if rem:
                pltpu.sync_copy(
                    rows_v.at[pl.ds(0, rem)],
                    acc_sh.at[pl.ds(r0 + (rows_per_zt // chunk) * chunk, rem)])

        plsc.subcore_barrier()

        def chunk_body(k, _):
            idx = src_v.at[pl.ds(k * chunk, chunk)]
            pltpu.async_copy(in_hbm.at[idx], rows_v, sem).wait()

            def scale(g, _):
                wv = w_v[pl.ds(k * chunk + g * _L, _L)]
                for e in range(_L):
                    we = wv[e]
                    row = g * _L + e
                    for j in range(d // _L):
                        sl = pl.ds(j * _L, _L)
                        rows_v[row, sl] = rows_v[row, sl] * we
                return 0

            lax.fori_loop(0, chunk // _L, scale, 0)
            pltpu.sync_copy(rows_v, acc_sh.at[dst_v.at[k]], add=True)
            return 0

        lax.fori_loop(0, n_chunks, chunk_body, 0)
        plsc.subcore_barrier()

        @pl.when(sid < nzt)
        def _writeback():
            pltpu.sync_copy(
                acc_sh.at[pl.ds(r0, rows_per_zt)],
                out_hbm.at[pl.ds(pl.multiple_of(cid * n_nodes + sid * rows_per_zt, 8),
                                 rows_per_zt)],
            )

    return layer


def _combine(partials, acc, n_nodes, d):
    br = 1000
    grid = n_nodes // br

    def body(p_ref, acc_ref, e_ref, accout_ref):
        e = p_ref[0] + p_ref[1]
        e_ref[...] = e
        accout_ref[...] = acc_ref[...] + e

    p3 = partials.reshape(2, n_nodes, d)
    return pl.pallas_call(
        body,
        grid=(grid,),
        in_specs=[
            pl.BlockSpec((2, br, d), lambda i: (0, i, 0)),
            pl.BlockSpec((br, d), lambda i: (i, 0)),
        ],
        out_specs=[
            pl.BlockSpec((br, d), lambda i: (i, 0)),
            pl.BlockSpec((br, d), lambda i: (i, 0)),
        ],
        out_shape=[jax.ShapeDtypeStruct((n_nodes, d), jnp.float32)] * 2,
    )(p3, acc)


def _combine_last(partials, acc, n_nodes, d):
    br = 1000
    grid = n_nodes // br

    def body(p_ref, acc_ref, accout_ref):
        accout_ref[...] = acc_ref[...] + p_ref[0] + p_ref[1]

    p3 = partials.reshape(2, n_nodes, d)
    return pl.pallas_call(
        body,
        grid=(grid,),
        in_specs=[
            pl.BlockSpec((2, br, d), lambda i: (0, i, 0)),
            pl.BlockSpec((br, d), lambda i: (i, 0)),
        ],
        out_specs=pl.BlockSpec((br, d), lambda i: (i, 0)),
        out_shape=jax.ShapeDtypeStruct((n_nodes, d), jnp.float32),
    )(p3, acc)


def _make_final(n_nodes, d, b, nc, ns):
    nw = nc * ns
    per_w = b // nw
    mesh = plsc.VectorSubcoreMesh(core_axis_name="c", subcore_axis_name="s")

    @functools.partial(
        pl.kernel,
        mesh=mesh,
        compiler_params=pltpu.CompilerParams(needs_layout_passes=False),
        out_type=jax.ShapeDtypeStruct((b,), jnp.float32),
        scratch_types=[
            pltpu.VMEM((per_w,), jnp.int32),
            pltpu.VMEM((per_w,), jnp.int32),
            pltpu.VMEM((per_w, d), jnp.float32),
            pltpu.VMEM((per_w, d), jnp.float32),
            pltpu.VMEM((per_w,), jnp.float32),
            pltpu.SemaphoreType.DMA,
        ],
    )
    def final(acc_hbm, users_hbm, items_hbm, out_hbm,
              ui_v, ii_v, ur_v, ir_v, g_v, sem):
        cid = lax.axis_index("c")
        sid = lax.axis_index("s")
        wid = sid * nc + cid
        b0 = pl.multiple_of(wid * per_w, 8)
        pltpu.sync_copy(users_hbm.at[pl.ds(b0, per_w)], ui_v)
        pltpu.sync_copy(items_hbm.at[pl.ds(b0, per_w)], ii_v)
        pltpu.async_copy(acc_hbm.at[ui_v], ur_v, sem).wait()
        pltpu.async_copy(acc_hbm.at[ii_v], ir_v, sem).wait()

        lane = lax.iota(jnp.int32, _L)

        def dot16(grp, _):
            rowi = lane + grp * _L

            def dloop(dd, acc16):
                cols = jnp.full((_L,), dd, jnp.int32)
                ut = plsc.load_gather(ur_v, [rowi, cols])
                it = plsc.load_gather(ir_v, [rowi, cols])
                return acc16 + ut * it

            acc16 = lax.fori_loop(0, d, dloop, jnp.zeros((_L,), jnp.float32))
            g_v[pl.ds(grp * _L, _L)] = acc16 * jnp.float32(1.0 / 16.0)
            return 0

        lax.fori_loop(0, per_w // _L, dot16, 0)
        pltpu.sync_copy(g_v, out_hbm.at[pl.ds(b0, per_w)])

    return final


def kernel(emb, edge_weight, edge_index, users, items):
    n_nodes, d = emb.shape
    n_edges = edge_weight.shape[0]
    b = users.shape[0]
    nc, ns = _sc_geometry()
    nw = nc * ns
    e_per_w = n_edges // nw
    chunk = 80
    n_chunks = e_per_w // chunk

    src = edge_index[0]
    dst = edge_index[1].reshape(nw, n_chunks, chunk)
    w = edge_weight

    layer = _make_layer(n_nodes, d, n_edges, nc, ns)
    final = _make_final(n_nodes, d, b, nc, ns)

    p1 = layer(emb, src, dst, w)
    e1, acc = _combine(p1, emb, n_nodes, d)
    p2 = layer(e1, src, dst, w)
    e2, acc = _combine(p2, acc, n_nodes, d)
    p3 = layer(e2, src, dst, w)
    acc = _combine_last(p3, acc, n_nodes, d)
    return final(acc, users, items)


# speedup vs baseline: 7.8811x; 1.4070x over previous
import functools

import jax
import jax.numpy as jnp
from jax import lax
from jax.experimental import pallas as pl
from jax.experimental.pallas import tpu as pltpu
from jax.experimental.pallas import tpu_sc as plsc


_L = 16


def _sc_geometry():
    info = plsc.get_sparse_core_info()
    return info.num_cores, info.num_subcores


def _make_layer(n_nodes, d, n_edges, nc, ns):
    nw = nc * ns
    e_per_w = n_edges // nw
    chunk = 80
    n_chunks = e_per_w // chunk
    nzt = 10
    rows_per_zt = n_nodes // nzt
    mesh = plsc.VectorSubcoreMesh(core_axis_name="c", subcore_axis_name="s")
    nsteady = ((n_chunks - 2) // 3) * 3

    @functools.partial(
        pl.kernel,
        mesh=mesh,
        out_type=jax.ShapeDtypeStruct((nc * n_nodes, d), jnp.float32),
        scratch_types=[
            pltpu.VMEM((e_per_w,), jnp.int32),
            pltpu.VMEM((chunk, d), jnp.float32),
            pltpu.VMEM((chunk, d), jnp.float32),
            pltpu.VMEM((chunk, d), jnp.float32),
            pltpu.VMEM((1, chunk), jnp.int32),
            pltpu.VMEM((1, chunk), jnp.int32),
            pltpu.VMEM((1, chunk), jnp.int32),
            pltpu.VMEM((chunk,), jnp.float32),
            pltpu.VMEM((chunk,), jnp.float32),
            pltpu.VMEM((chunk,), jnp.float32),
            pltpu.VMEM_SHARED((n_nodes, d), jnp.float32),
        ] + [pltpu.SemaphoreType.DMA] * 12,
    )
    def layer(in_hbm, src_hbm, dst_hbm, w_hbm, out_hbm,
              src_v, rows0, rows1, rows2, dstb0, dstb1, dstb2,
              wb0, wb1, wb2, acc_sh,
              sg0, sg1, sg2, ss0, ss1, ss2, sd0, sd1, sd2, sw0, sw1, sw2):
        rows = (rows0, rows1, rows2)
        dstb = (dstb0, dstb1, dstb2)
        wb = (wb0, wb1, wb2)
        sg = (sg0, sg1, sg2)
        ss = (ss0, ss1, ss2)
        sd = (sd0, sd1, sd2)
        sw = (sw0, sw1, sw2)
        cid = lax.axis_index("c")
        sid = lax.axis_index("s")
        wid = sid * nc + cid
        ebase = pl.multiple_of(wid * e_per_w, 8)

        pltpu.sync_copy(src_hbm.at[pl.ds(ebase, e_per_w)], src_v)

        def issue_gather(kk, j):
            idx = src_v.at[pl.ds(pl.multiple_of(kk * chunk, 8), chunk)]
            pltpu.async_copy(in_hbm.at[idx], rows[j], sg[j])

        def wait_gather(kk, j):
            idx = src_v.at[pl.ds(pl.multiple_of(kk * chunk, 8), chunk)]
            pltpu.make_async_copy(in_hbm.at[idx], rows[j], sg[j]).wait()

        def issue_dw(kk, j):
            off = pl.ds(pl.multiple_of(ebase + kk * chunk, 8), chunk)
            pltpu.async_copy(dst_hbm.at[off], dstb[j].at[0], sd[j])
            pltpu.async_copy(w_hbm.at[off], wb[j], sw[j])

        def wait_dw(kk, j):
            off = pl.ds(pl.multiple_of(ebase + kk * chunk, 8), chunk)
            pltpu.make_async_copy(dst_hbm.at[off], dstb[j].at[0], sd[j]).wait()
            pltpu.make_async_copy(w_hbm.at[off], wb[j], sw[j]).wait()

        def issue_scatter(j):
            pltpu.async_copy(rows[j], acc_sh.at[dstb[j].at[0]], ss[j],
                             add=True)

        def wait_scatter(j):
            pltpu.make_async_copy(rows[j], acc_sh.at[dstb[j].at[0]],
                                  ss[j]).wait()

        issue_dw(0, 0)
        issue_dw(1, 1)
        issue_gather(0, 0)
        issue_gather(1, 1)

        zvec = jnp.zeros((_L,), jnp.float32)

        def zfill(i, _):
            for j in range(d // _L):
                rows2[i, pl.ds(j * _L, _L)] = zvec
            return 0

        lax.fori_loop(0, chunk, zfill, 0)
        r0 = pl.multiple_of(sid * rows_per_zt, 8)

        @pl.when(sid < nzt)
        def _zero():
            for z in range(rows_per_zt // chunk):
                pltpu.sync_copy(rows2,
                                acc_sh.at[pl.ds(r0 + z * chunk, chunk)])
            rem = rows_per_zt % chunk
            if rem:
                pltpu.sync_copy(
                    rows2.at[pl.ds(0, rem)],
                    acc_sh.at[pl.ds(r0 + (rows_per_zt // chunk) * chunk, rem)])

        plsc.subcore_barrier()

        def scale(j):
            for g in range(chunk // _L):
                wv = wb[j][pl.ds(g * _L, _L)]
                for e in range(_L):
                    we = wv[e]
                    row = g * _L + e
                    for jj in range(d // _L):
                        sl = pl.ds(jj * _L, _L)
                        rows[j][row, sl] = rows[j][row, sl] * we

        def template(kk, j, prepare):
            if prepare:
                j2 = (j + 2) % 3

                @pl.when(kk >= 1)
                def _():
                    wait_scatter(j2)

                issue_dw(kk + 2, j2)
                issue_gather(kk + 2, j2)
            wait_gather(kk, j)
            wait_dw(kk, j)
            scale(j)
            issue_scatter(j)

        @pl.loop(0, nsteady, step=3)
        def _steady(kbase):
            for i in range(3):
                template(kbase + i, i, True)

        for kk in range(nsteady, n_chunks):
            template(kk, kk % 3, False)

        for kk in range(n_chunks - 3, n_chunks):
            wait_scatter(kk % 3)

        plsc.subcore_barrier()

        @pl.when(sid < nzt)
        def _writeback():
            pltpu.sync_copy(
                acc_sh.at[pl.ds(r0, rows_per_zt)],
                out_hbm.at[pl.ds(pl.multiple_of(cid * n_nodes + sid * rows_per_zt, 8),
                                 rows_per_zt)],
            )

    return layer


def _combine(partials, acc, n_nodes, d):
    br = 1000
    grid = n_nodes // br

    def body(p_ref, acc_ref, e_ref, accout_ref):
        e = p_ref[0] + p_ref[1]
        e_ref[...] = e
        accout_ref[...] = acc_ref[...] + e

    p3 = partials.reshape(2, n_nodes, d)
    return pl.pallas_call(
        body,
        grid=(grid,),
        in_specs=[
            pl.BlockSpec((2, br, d), lambda i: (0, i, 0)),
            pl.BlockSpec((br, d), lambda i: (i, 0)),
        ],
        out_specs=[
            pl.BlockSpec((br, d), lambda i: (i, 0)),
            pl.BlockSpec((br, d), lambda i: (i, 0)),
        ],
        out_shape=[jax.ShapeDtypeStruct((n_nodes, d), jnp.float32)] * 2,
    )(p3, acc)


def _combine_last(partials, acc, n_nodes, d):
    br = 1000
    grid = n_nodes // br

    def body(p_ref, acc_ref, accout_ref):
        accout_ref[...] = acc_ref[...] + p_ref[0] + p_ref[1]

    p3 = partials.reshape(2, n_nodes, d)
    return pl.pallas_call(
        body,
        grid=(grid,),
        in_specs=[
            pl.BlockSpec((2, br, d), lambda i: (0, i, 0)),
            pl.BlockSpec((br, d), lambda i: (i, 0)),
        ],
        out_specs=pl.BlockSpec((br, d), lambda i: (i, 0)),
        out_shape=jax.ShapeDtypeStruct((n_nodes, d), jnp.float32),
    )(p3, acc)


def _make_final(n_nodes, d, b, nc, ns):
    nw = nc * ns
    per_w = b // nw
    mesh = plsc.VectorSubcoreMesh(core_axis_name="c", subcore_axis_name="s")

    @functools.partial(
        pl.kernel,
        mesh=mesh,
        compiler_params=pltpu.CompilerParams(needs_layout_passes=False),
        out_type=jax.ShapeDtypeStruct((b,), jnp.float32),
        scratch_types=[
            pltpu.VMEM((per_w,), jnp.int32),
            pltpu.VMEM((per_w,), jnp.int32),
            pltpu.VMEM((per_w, d), jnp.float32),
            pltpu.VMEM((per_w, d), jnp.float32),
            pltpu.VMEM((per_w,), jnp.float32),
            pltpu.SemaphoreType.DMA,
        ],
    )
    def final(acc_hbm, users_hbm, items_hbm, out_hbm,
              ui_v, ii_v, ur_v, ir_v, g_v, sem):
        cid = lax.axis_index("c")
        sid = lax.axis_index("s")
        wid = sid * nc + cid
        b0 = pl.multiple_of(wid * per_w, 8)
        pltpu.sync_copy(users_hbm.at[pl.ds(b0, per_w)], ui_v)
        pltpu.sync_copy(items_hbm.at[pl.ds(b0, per_w)], ii_v)
        pltpu.async_copy(acc_hbm.at[ui_v], ur_v, sem).wait()
        pltpu.async_copy(acc_hbm.at[ii_v], ir_v, sem).wait()

        lane = lax.iota(jnp.int32, _L)

        def dot16(grp, _):
            rowi = lane + grp * _L

            def dloop(dd, acc16):
                cols = jnp.full((_L,), dd, jnp.int32)
                ut = plsc.load_gather(ur_v, [rowi, cols])
                it = plsc.load_gather(ir_v, [rowi, cols])
                return acc16 + ut * it

            acc16 = lax.fori_loop(0, d, dloop, jnp.zeros((_L,), jnp.float32))
            g_v[pl.ds(grp * _L, _L)] = acc16 * jnp.float32(1.0 / 16.0)
            return 0

        lax.fori_loop(0, per_w // _L, dot16, 0)
        pltpu.sync_copy(g_v, out_hbm.at[pl.ds(b0, per_w)])

    return final


def kernel(emb, edge_weight, edge_index, users, items):
    n_nodes, d = emb.shape
    n_edges = edge_weight.shape[0]
    b = users.shape[0]
    nc, ns = _sc_geometry()
    nw = nc * ns
    e_per_w = n_edges // nw
    chunk = 80
    n_chunks = e_per_w // chunk

    src = edge_index[0]
    dst = edge_index[1]
    w = edge_weight

    layer = _make_layer(n_nodes, d, n_edges, nc, ns)
    final = _make_final(n_nodes, d, b, nc, ns)

    p1 = layer(emb, src, dst, w)
    e1, acc = _combine(p1, emb, n_nodes, d)
    p2 = layer(e1, src, dst, w)
    e2, acc = _combine(p2, acc, n_nodes, d)
    p3 = layer(e2, src, dst, w)
    acc = _combine_last(p3, acc, n_nodes, d)
    return final(acc, users, items)


# speedup vs baseline: 7.8917x; 1.0013x over previous
import functools

import jax
import jax.numpy as jnp
from jax import lax
from jax.experimental import pallas as pl
from jax.experimental.pallas import tpu as pltpu
from jax.experimental.pallas import tpu_sc as plsc


_L = 16


def _sc_geometry():
    info = plsc.get_sparse_core_info()
    return info.num_cores, info.num_subcores


def _make_layer(n_nodes, d, n_edges, nc, ns):
    nw = nc * ns
    e_per_w = n_edges // nw
    chunk = 80
    n_chunks = e_per_w // chunk
    nzt = 10
    rows_per_zt = n_nodes // nzt
    mesh = plsc.VectorSubcoreMesh(core_axis_name="c", subcore_axis_name="s")
    nsteady = ((n_chunks - 2) // 3) * 3

    @functools.partial(
        pl.kernel,
        mesh=mesh,
        out_type=jax.ShapeDtypeStruct((nc * n_nodes, d), jnp.float32),
        scratch_types=[
            pltpu.VMEM((e_per_w,), jnp.int32),
            pltpu.VMEM((chunk, d), jnp.float32),
            pltpu.VMEM((chunk, d), jnp.float32),
            pltpu.VMEM((chunk, d), jnp.float32),
            pltpu.VMEM((1, chunk), jnp.int32),
            pltpu.VMEM((1, chunk), jnp.int32),
            pltpu.VMEM((1, chunk), jnp.int32),
            pltpu.VMEM((chunk,), jnp.float32),
            pltpu.VMEM((chunk,), jnp.float32),
            pltpu.VMEM((chunk,), jnp.float32),
            pltpu.VMEM_SHARED((n_nodes, d), jnp.float32),
        ] + [pltpu.SemaphoreType.DMA] * 12,
    )
    def layer(in_hbm, src_hbm, dst_hbm, w_hbm, out_hbm,
              src_v, rows0, rows1, rows2, dstb0, dstb1, dstb2,
              wb0, wb1, wb2, acc_sh,
              sg0, sg1, sg2, ss0, ss1, ss2, sd0, sd1, sd2, sw0, sw1, sw2):
        rows = (rows0, rows1, rows2)
        dstb = (dstb0, dstb1, dstb2)
        wb = (wb0, wb1, wb2)
        sg = (sg0, sg1, sg2)
        ss = (ss0, ss1, ss2)
        sd = (sd0, sd1, sd2)
        sw = (sw0, sw1, sw2)
        cid = lax.axis_index("c")
        sid = lax.axis_index("s")
        wid = sid * nc + cid
        ebase = pl.multiple_of(wid * e_per_w, 8)

        pltpu.sync_copy(src_hbm.at[pl.ds(ebase, e_per_w)], src_v)

        def issue_gather(kk, j):
            idx = src_v.at[pl.ds(pl.multiple_of(kk * chunk, 8), chunk)]
            pltpu.async_copy(in_hbm.at[idx], rows[j], sg[j])

        def wait_gather(kk, j):
            idx = src_v.at[pl.ds(pl.multiple_of(kk * chunk, 8), chunk)]
            pltpu.make_async_copy(in_hbm.at[idx], rows[j], sg[j]).wait()

        def issue_dw(kk, j):
            off = pl.ds(pl.multiple_of(ebase + kk * chunk, 8), chunk)
            pltpu.async_copy(dst_hbm.at[off], dstb[j].at[0], sd[j])
            pltpu.async_copy(w_hbm.at[off], wb[j], sw[j])

        def wait_dw(kk, j):
            off = pl.ds(pl.multiple_of(ebase + kk * chunk, 8), chunk)
            pltpu.make_async_copy(dst_hbm.at[off], dstb[j].at[0], sd[j]).wait()
            pltpu.make_async_copy(w_hbm.at[off], wb[j], sw[j]).wait()

        def issue_scatter(j):
            pltpu.async_copy(rows[j], acc_sh.at[dstb[j].at[0]], ss[j],
                             add=True)

        def wait_scatter(j):
            pltpu.make_async_copy(rows[j], acc_sh.at[dstb[j].at[0]],
                                  ss[j]).wait()

        issue_dw(0, 0)
        issue_dw(1, 1)
        issue_gather(0, 0)
        issue_gather(1, 1)

        zvec = jnp.zeros((_L,), jnp.float32)

        def zfill(i, _):
            for j in range(d // _L):
                rows2[i, pl.ds(j * _L, _L)] = zvec
            return 0

        lax.fori_loop(0, chunk, zfill, 0)
        r0 = pl.multiple_of(sid * rows_per_zt, 8)

        @pl.when(sid < nzt)
        def _zero():
            for z in range(rows_per_zt // chunk):
                pltpu.sync_copy(rows2,
                                acc_sh.at[pl.ds(r0 + z * chunk, chunk)])
            rem = rows_per_zt % chunk
            if rem:
                pltpu.sync_copy(
                    rows2.at[pl.ds(0, rem)],
                    acc_sh.at[pl.ds(r0 + (rows_per_zt // chunk) * chunk, rem)])

        plsc.subcore_barrier()

        def scale(j):
            for g in range(chunk // _L):
                wv = wb[j][pl.ds(g * _L, _L)]
                for e in range(_L):
                    we = wv[e]
                    row = g * _L + e
                    for jj in range(d // _L):
                        sl = pl.ds(jj * _L, _L)
                        rows[j][row, sl] = rows[j][row, sl] * we

        def template(kk, j, prepare):
            if prepare:
                j2 = (j + 2) % 3

                @pl.when(kk >= 1)
                def _():
                    wait_scatter(j2)

                issue_dw(kk + 2, j2)
                issue_gather(kk + 2, j2)
            wait_gather(kk, j)
            wait_dw(kk, j)
            scale(j)
            issue_scatter(j)

        @pl.loop(0, nsteady, step=3)
        def _steady(kbase):
            for i in range(3):
                template(kbase + i, i, True)

        for kk in range(nsteady, n_chunks):
            template(kk, kk % 3, False)

        for kk in range(n_chunks - 3, n_chunks):
            wait_scatter(kk % 3)

        plsc.subcore_barrier()

        @pl.when(sid < nzt)
        def _writeback():
            pltpu.sync_copy(
                acc_sh.at[pl.ds(r0, rows_per_zt)],
                out_hbm.at[pl.ds(pl.multiple_of(cid * n_nodes + sid * rows_per_zt, 8),
                                 rows_per_zt)],
            )

    return layer


def _esum(partials, n_nodes, d):
    br = 1000
    grid = n_nodes // br

    def body(p_ref, e_ref):
        e_ref[...] = p_ref[0] + p_ref[1]

    p3 = partials.reshape(2, n_nodes, d)
    return pl.pallas_call(
        body,
        grid=(grid,),
        in_specs=[pl.BlockSpec((2, br, d), lambda i: (0, i, 0))],
        out_specs=pl.BlockSpec((br, d), lambda i: (i, 0)),
        out_shape=jax.ShapeDtypeStruct((n_nodes, d), jnp.float32),
    )(p3)


def _accadd(acc, e, n_nodes, d):
    br = 1000
    grid = n_nodes // br

    def body(acc_ref, e_ref, out_ref):
        out_ref[...] = acc_ref[...] + e_ref[...]

    return pl.pallas_call(
        body,
        grid=(grid,),
        in_specs=[
            pl.BlockSpec((br, d), lambda i: (i, 0)),
            pl.BlockSpec((br, d), lambda i: (i, 0)),
        ],
        out_specs=pl.BlockSpec((br, d), lambda i: (i, 0)),
        out_shape=jax.ShapeDtypeStruct((n_nodes, d), jnp.float32),
    )(acc, e)


def _combine_last(partials, acc, n_nodes, d):
    br = 1000
    grid = n_nodes // br

    def body(p_ref, acc_ref, accout_ref):
        accout_ref[...] = acc_ref[...] + p_ref[0] + p_ref[1]

    p3 = partials.reshape(2, n_nodes, d)
    return pl.pallas_call(
        body,
        grid=(grid,),
        in_specs=[
            pl.BlockSpec((2, br, d), lambda i: (0, i, 0)),
            pl.BlockSpec((br, d), lambda i: (i, 0)),
        ],
        out_specs=pl.BlockSpec((br, d), lambda i: (i, 0)),
        out_shape=jax.ShapeDtypeStruct((n_nodes, d), jnp.float32),
    )(p3, acc)


def _make_final(n_nodes, d, b, nc, ns):
    nw = nc * ns
    per_w = b // nw
    mesh = plsc.VectorSubcoreMesh(core_axis_name="c", subcore_axis_name="s")

    @functools.partial(
        pl.kernel,
        mesh=mesh,
        compiler_params=pltpu.CompilerParams(needs_layout_passes=False),
        out_type=jax.ShapeDtypeStruct((b,), jnp.float32),
        scratch_types=[
            pltpu.VMEM((per_w,), jnp.int32),
            pltpu.VMEM((per_w,), jnp.int32),
            pltpu.VMEM((per_w, d), jnp.float32),
            pltpu.VMEM((per_w, d), jnp.float32),
            pltpu.VMEM((per_w,), jnp.float32),
            pltpu.SemaphoreType.DMA,
        ],
    )
    def final(acc_hbm, users_hbm, items_hbm, out_hbm,
              ui_v, ii_v, ur_v, ir_v, g_v, sem):
        cid = lax.axis_index("c")
        sid = lax.axis_index("s")
        wid = sid * nc + cid
        b0 = pl.multiple_of(wid * per_w, 8)
        pltpu.sync_copy(users_hbm.at[pl.ds(b0, per_w)], ui_v)
        pltpu.sync_copy(items_hbm.at[pl.ds(b0, per_w)], ii_v)
        pltpu.async_copy(acc_hbm.at[ui_v], ur_v, sem).wait()
        pltpu.async_copy(acc_hbm.at[ii_v], ir_v, sem).wait()

        lane = lax.iota(jnp.int32, _L)

        def dot16(grp, _):
            rowi = lane + grp * _L

            def dloop(dd, acc16):
                cols = jnp.full((_L,), dd, jnp.int32)
                ut = plsc.load_gather(ur_v, [rowi, cols])
                it = plsc.load_gather(ir_v, [rowi, cols])
                return acc16 + ut * it

            acc16 = lax.fori_loop(0, d, dloop, jnp.zeros((_L,), jnp.float32))
            g_v[pl.ds(grp * _L, _L)] = acc16 * jnp.float32(1.0 / 16.0)
            return 0

        lax.fori_loop(0, per_w // _L, dot16, 0)
        pltpu.sync_copy(g_v, out_hbm.at[pl.ds(b0, per_w)])

    return final


def kernel(emb, edge_weight, edge_index, users, items):
    n_nodes, d = emb.shape
    n_edges = edge_weight.shape[0]
    b = users.shape[0]
    nc, ns = _sc_geometry()

    src = edge_index[0]
    dst = edge_index[1]
    w = edge_weight

    layer = _make_layer(n_nodes, d, n_edges, nc, ns)
    final = _make_final(n_nodes, d, b, nc, ns)

    p1 = layer(emb, src, dst, w)
    e1 = _esum(p1, n_nodes, d)
    acc = _accadd(emb, e1, n_nodes, d)
    p2 = layer(e1, src, dst, w)
    e2 = _esum(p2, n_nodes, d)
    acc = _accadd(acc, e2, n_nodes, d)
    p3 = layer(e2, src, dst, w)
    acc = _combine_last(p3, acc, n_nodes, d)
    return final(acc, users, items)


# speedup vs baseline: 8.0168x; 1.0159x over previous
import functools

import jax
import jax.numpy as jnp
from jax import lax
from jax.experimental import pallas as pl
from jax.experimental.pallas import tpu as pltpu
from jax.experimental.pallas import tpu_sc as plsc


_L = 16


def _sc_geometry():
    info = plsc.get_sparse_core_info()
    return info.num_cores, info.num_subcores


def _make_layer(n_nodes, d, n_edges, nc, ns):
    nw = nc * ns
    e_per_w = n_edges // nw
    chunk = 80
    n_chunks = e_per_w // chunk
    nzt = 10
    rows_per_zt = n_nodes // nzt
    mesh = plsc.VectorSubcoreMesh(core_axis_name="c", subcore_axis_name="s")
    nsteady = ((n_chunks - 2) // 3) * 3

    @functools.partial(
        pl.kernel,
        mesh=mesh,
        out_type=jax.ShapeDtypeStruct((nc * n_nodes, d), jnp.float32),
        scratch_types=[
            pltpu.VMEM((e_per_w,), jnp.int32),
            pltpu.VMEM((chunk, d), jnp.float32),
            pltpu.VMEM((chunk, d), jnp.float32),
            pltpu.VMEM((chunk, d), jnp.float32),
            pltpu.VMEM((1, chunk), jnp.int32),
            pltpu.VMEM((1, chunk), jnp.int32),
            pltpu.VMEM((1, chunk), jnp.int32),
            pltpu.VMEM((chunk,), jnp.float32),
            pltpu.VMEM((chunk,), jnp.float32),
            pltpu.VMEM((chunk,), jnp.float32),
            pltpu.VMEM_SHARED((n_nodes, d), jnp.float32),
        ] + [pltpu.SemaphoreType.DMA] * 12,
    )
    def layer(in_hbm, src_hbm, dst_hbm, w_hbm, out_hbm,
              src_v, rows0, rows1, rows2, dstb0, dstb1, dstb2,
              wb0, wb1, wb2, acc_sh,
              sg0, sg1, sg2, ss0, ss1, ss2, sd0, sd1, sd2, sw0, sw1, sw2):
        rows = (rows0, rows1, rows2)
        dstb = (dstb0, dstb1, dstb2)
        wb = (wb0, wb1, wb2)
        sg = (sg0, sg1, sg2)
        ss = (ss0, ss1, ss2)
        sd = (sd0, sd1, sd2)
        sw = (sw0, sw1, sw2)
        cid = lax.axis_index("c")
        sid = lax.axis_index("s")
        wid = sid * nc + cid
        ebase = pl.multiple_of(wid * e_per_w, 8)

        pltpu.sync_copy(src_hbm.at[pl.ds(ebase, e_per_w)], src_v)

        def issue_gather(kk, j):
            idx = src_v.at[pl.ds(pl.multiple_of(kk * chunk, 8), chunk)]
            pltpu.async_copy(in_hbm.at[idx], rows[j], sg[j])

        def wait_gather(kk, j):
            idx = src_v.at[pl.ds(pl.multiple_of(kk * chunk, 8), chunk)]
            pltpu.make_async_copy(in_hbm.at[idx], rows[j], sg[j]).wait()

        def issue_dw(kk, j):
            off = pl.ds(pl.multiple_of(ebase + kk * chunk, 8), chunk)
            pltpu.async_copy(dst_hbm.at[off], dstb[j].at[0], sd[j])
            pltpu.async_copy(w_hbm.at[off], wb[j], sw[j])

        def wait_dw(kk, j):
            off = pl.ds(pl.multiple_of(ebase + kk * chunk, 8), chunk)
            pltpu.make_async_copy(dst_hbm.at[off], dstb[j].at[0], sd[j]).wait()
            pltpu.make_async_copy(w_hbm.at[off], wb[j], sw[j]).wait()

        def issue_scatter(j):
            pltpu.async_copy(rows[j], acc_sh.at[dstb[j].at[0]], ss[j],
                             add=True)

        def wait_scatter(j):
            pltpu.make_async_copy(rows[j], acc_sh.at[dstb[j].at[0]],
                                  ss[j]).wait()

        issue_dw(0, 0)
        issue_dw(1, 1)
        issue_gather(0, 0)
        issue_gather(1, 1)

        zvec = jnp.zeros((_L,), jnp.float32)

        def zfill(i, _):
            for j in range(d // _L):
                rows2[i, pl.ds(j * _L, _L)] = zvec
            return 0

        lax.fori_loop(0, chunk, zfill, 0)
        r0 = pl.multiple_of(sid * rows_per_zt, 8)

        @pl.when(sid < nzt)
        def _zero():
            for z in range(rows_per_zt // chunk):
                pltpu.sync_copy(rows2,
                                acc_sh.at[pl.ds(r0 + z * chunk, chunk)])
            rem = rows_per_zt % chunk
            if rem:
                pltpu.sync_copy(
                    rows2.at[pl.ds(0, rem)],
                    acc_sh.at[pl.ds(r0 + (rows_per_zt // chunk) * chunk, rem)])

        plsc.subcore_barrier()

        def scale(j):
            for g in range(chunk // _L):
                wv = wb[j][pl.ds(g * _L, _L)]
                for e in range(_L):
                    we = wv[e]
                    row = g * _L + e
                    for jj in range(d // _L):
                        sl = pl.ds(jj * _L, _L)
                        rows[j][row, sl] = rows[j][row, sl] * we

        def template(kk, j, prepare):
            if prepare:
                j2 = (j + 2) % 3

                @pl.when(kk >= 1)
                def _():
                    wait_scatter(j2)

                issue_dw(kk + 2, j2)
                issue_gather(kk + 2, j2)
            wait_gather(kk, j)
            wait_dw(kk, j)
            scale(j)
            issue_scatter(j)

        @pl.loop(0, nsteady, step=3)
        def _steady(kbase):
            for i in range(3):
                template(kbase + i, i, True)

        for kk in range(nsteady, n_chunks):
            template(kk, kk % 3, False)

        for kk in range(n_chunks - 3, n_chunks):
            wait_scatter(kk % 3)

        plsc.subcore_barrier()

        @pl.when(sid < nzt)
        def _writeback():
            pltpu.sync_copy(
                acc_sh.at[pl.ds(r0, rows_per_zt)],
                out_hbm.at[pl.ds(pl.multiple_of(cid * n_nodes + sid * rows_per_zt, 8),
                                 rows_per_zt)],
            )

    return layer


def _esum(partials, n_nodes, d):
    br = 1000
    grid = n_nodes // br

    def body(p_ref, e_ref):
        e_ref[...] = p_ref[0] + p_ref[1]

    p3 = partials.reshape(2, n_nodes, d)
    return pl.pallas_call(
        body,
        grid=(grid,),
        in_specs=[pl.BlockSpec((2, br, d), lambda i: (0, i, 0))],
        out_specs=pl.BlockSpec((br, d), lambda i: (i, 0)),
        out_shape=jax.ShapeDtypeStruct((n_nodes, d), jnp.float32),
    )(p3)


def _accadd(acc, e, n_nodes, d):
    br = 1000
    grid = n_nodes // br

    def body(acc_ref, e_ref, out_ref):
        out_ref[...] = acc_ref[...] + e_ref[...]

    return pl.pallas_call(
        body,
        grid=(grid,),
        in_specs=[
            pl.BlockSpec((br, d), lambda i: (i, 0)),
            pl.BlockSpec((br, d), lambda i: (i, 0)),
        ],
        out_specs=pl.BlockSpec((br, d), lambda i: (i, 0)),
        out_shape=jax.ShapeDtypeStruct((n_nodes, d), jnp.float32),
    )(acc, e)


def _make_final(n_nodes, d, b, nc, ns):
    nw = nc * ns
    per_w = b // nw
    mesh = plsc.VectorSubcoreMesh(core_axis_name="c", subcore_axis_name="s")

    @functools.partial(
        pl.kernel,
        mesh=mesh,
        compiler_params=pltpu.CompilerParams(needs_layout_passes=False),
        out_type=jax.ShapeDtypeStruct((b,), jnp.float32),
        scratch_types=[
            pltpu.VMEM((per_w,), jnp.int32),
            pltpu.VMEM((per_w,), jnp.int32),
            pltpu.VMEM((per_w,), jnp.int32),
            pltpu.VMEM((per_w,), jnp.int32),
            pltpu.VMEM((per_w, d), jnp.float32),
            pltpu.VMEM((per_w, d), jnp.float32),
            pltpu.VMEM((per_w, d), jnp.float32),
            pltpu.VMEM((per_w, d), jnp.float32),
            pltpu.VMEM((per_w, d), jnp.float32),
            pltpu.VMEM((per_w, d), jnp.float32),
            pltpu.VMEM((per_w,), jnp.float32),
        ] + [pltpu.SemaphoreType.DMA] * 6,
    )
    def final(acc_hbm, p3_hbm, users_hbm, items_hbm, out_hbm,
              ui_v, ii_v, ui2_v, ii2_v, ur_v, ir_v, ua_v, ia_v, ub_v, ib_v,
              g_v, s0, s1, s2, s3, s4, s5):
        cid = lax.axis_index("c")
        sid = lax.axis_index("s")
        wid = sid * nc + cid
        b0 = pl.multiple_of(wid * per_w, 8)
        pltpu.sync_copy(users_hbm.at[pl.ds(b0, per_w)], ui_v)
        pltpu.sync_copy(items_hbm.at[pl.ds(b0, per_w)], ii_v)
        nvec = jnp.full((_L,), n_nodes, jnp.int32)
        for g in range(per_w // _L):
            sl = pl.ds(g * _L, _L)
            ui2_v[sl] = ui_v[sl] + nvec
            ii2_v[sl] = ii_v[sl] + nvec
        pltpu.async_copy(acc_hbm.at[ui_v], ur_v, s0)
        pltpu.async_copy(acc_hbm.at[ii_v], ir_v, s1)
        pltpu.async_copy(p3_hbm.at[ui_v], ua_v, s2)
        pltpu.async_copy(p3_hbm.at[ii_v], ia_v, s3)
        pltpu.async_copy(p3_hbm.at[ui2_v], ub_v, s4)
        pltpu.async_copy(p3_hbm.at[ii2_v], ib_v, s5)
        pltpu.make_async_copy(acc_hbm.at[ui_v], ur_v, s0).wait()
        pltpu.make_async_copy(acc_hbm.at[ii_v], ir_v, s1).wait()
        pltpu.make_async_copy(p3_hbm.at[ui_v], ua_v, s2).wait()
        pltpu.make_async_copy(p3_hbm.at[ii_v], ia_v, s3).wait()
        pltpu.make_async_copy(p3_hbm.at[ui2_v], ub_v, s4).wait()
        pltpu.make_async_copy(p3_hbm.at[ii2_v], ib_v, s5).wait()

        def rsum(i, _):
            for j in range(d // _L):
                sl = pl.ds(j * _L, _L)
                ur_v[i, sl] = ur_v[i, sl] + ua_v[i, sl] + ub_v[i, sl]
                ir_v[i, sl] = ir_v[i, sl] + ia_v[i, sl] + ib_v[i, sl]
            return 0

        lax.fori_loop(0, per_w, rsum, 0)

        lane = lax.iota(jnp.int32, _L)

        def dot16(grp, _):
            rowi = lane + grp * _L

            def dloop(dd, acc16):
                cols = jnp.full((_L,), dd, jnp.int32)
                ut = plsc.load_gather(ur_v, [rowi, cols])
                it = plsc.load_gather(ir_v, [rowi, cols])
                return acc16 + ut * it

            acc16 = lax.fori_loop(0, d, dloop, jnp.zeros((_L,), jnp.float32))
            g_v[pl.ds(grp * _L, _L)] = acc16 * jnp.float32(1.0 / 16.0)
            return 0

        lax.fori_loop(0, per_w // _L, dot16, 0)
        pltpu.sync_copy(g_v, out_hbm.at[pl.ds(b0, per_w)])

    return final


def kernel(emb, edge_weight, edge_index, users, items):
    n_nodes, d = emb.shape
    n_edges = edge_weight.shape[0]
    b = users.shape[0]
    nc, ns = _sc_geometry()

    src = edge_index[0]
    dst = edge_index[1]
    w = edge_weight

    layer = _make_layer(n_nodes, d, n_edges, nc, ns)
    final = _make_final(n_nodes, d, b, nc, ns)

    p1 = layer(emb, src, dst, w)
    e1 = _esum(p1, n_nodes, d)
    acc = _accadd(emb, e1, n_nodes, d)
    p2 = layer(e1, src, dst, w)
    e2 = _esum(p2, n_nodes, d)
    acc = _accadd(acc, e2, n_nodes, d)
    p3 = layer(e2, src, dst, w)
    return final(acc, p3, users, items)
